# Initial kernel scaffold; baseline (speedup 1.0000x reference)
import jax
import jax.numpy as jnp
from jax.experimental import pallas as pl


def kernel(x, W, edge_src, edge_rel, edge_dst):
    raise NotImplementedError("write your pallas kernel here")


# speedup vs baseline: 4.6724x; 4.6724x over previous
import functools

import jax
import jax.numpy as jnp
from jax import lax
from jax.experimental import pallas as pl
from jax.experimental.pallas import tpu as pltpu
from jax.experimental.pallas import tpu_sc as plsc

N = 10000
R = 8
EMB = 128
E = 320000

NSUB = 16
NCORE = 2
CHUNK = 128
K = 256
QB = K // CHUNK

E_PAD = 327680
DUMMY = N
NPAD = 10240
CTBL = R * NPAD

NB1 = E_PAD // NSUB // K
NB2 = E_PAD // NCORE // NSUB // K

TROWS = NPAD // NSUB
CSTRIPE = CTBL // NSUB


def _mm_body(x_ref, w_ref, y_ref):
    y_ref[...] = lax.dot_general(
        x_ref[...], w_ref[0],
        dimension_numbers=(((1,), (1,)), ((), ())),
        preferred_element_type=jnp.float32)


def _relu_add_body(a_ref, b_ref, o_ref):
    o_ref[...] = jnp.maximum(a_ref[...] + b_ref[...], 0.0)


def _sc_body(y_hbm, src_hbm, rel_hbm, dst_hbm, zacc_hbm, zcnt_hbm, out_hbm,
             sbuf, rbuf, dbuf, gidx2, fr2, src2, cnt2, inv1, rows, ones,
             acc, cnt, sem):
    c = lax.axis_index("c")
    s = lax.axis_index("s")

    pltpu.sync_copy(zacc_hbm.at[pl.ds(s * TROWS, TROWS)],
                    acc.at[pl.ds(s * TROWS, TROWS)])
    pltpu.sync_copy(zcnt_hbm.at[pl.ds(s * CSTRIPE, CSTRIPE)],
                    cnt.at[pl.ds(s * CSTRIPE, CSTRIPE)])
    for g in range(CHUNK // 16):
        ones[pl.ds(g * 16, 16)] = jnp.full((16,), 1.0, dtype=jnp.float32)
    plsc.subcore_barrier()

    def p1_body(b, carry):
        base = (s * NB1 + b) * K
        pltpu.sync_copy(src_hbm.at[pl.ds(base, K)], sbuf)
        pltpu.sync_copy(rel_hbm.at[pl.ds(base, K)], rbuf)
        for g in range(K // 16):
            sv = sbuf[pl.ds(g * 16, 16)]
            rv = rbuf[pl.ds(g * 16, 16)]
            fr2[g // 8, pl.ds((g % 8) * 16, 16)] = rv * NPAD + sv
        for q in range(QB):
            pltpu.sync_copy(ones, cnt.at[fr2.at[q]], add=True)
        return carry

    lax.fori_loop(0, NB1, p1_body, 0)
    plsc.subcore_barrier()

    def p2_body(b, carry):
        base = c * (E_PAD // 2) + (s * NB2 + b) * K
        pltpu.sync_copy(src_hbm.at[pl.ds(base, K)], sbuf)
        pltpu.sync_copy(rel_hbm.at[pl.ds(base, K)], rbuf)
        pltpu.sync_copy(dst_hbm.at[pl.ds(base, K)], dbuf)
        for g in range(K // 16):
            sv = sbuf[pl.ds(g * 16, 16)]
            rv = rbuf[pl.ds(g * 16, 16)]
            dv = dbuf[pl.ds(g * 16, 16)]
            q, o = g // 8, (g % 8) * 16
            gidx2[q, pl.ds(o, 16)] = rv * N + dv
            fr2[q, pl.ds(o, 16)] = rv * NPAD + sv
            src2[q, pl.ds(o, 16)] = sv
        for q in range(QB):
            pltpu.sync_copy(y_hbm.at[gidx2.at[q]],
                            rows.at[pl.ds(q * CHUNK, CHUNK)])
            pltpu.sync_copy(cnt.at[fr2.at[q]], cnt2.at[q])
        for g in range(K // 16):
            cv = cnt2[g // 8, pl.ds((g % 8) * 16, 16)]
            inv1[pl.ds(g * 16, 16)] = 1.0 / cv
        dnums = lax.GatherDimensionNumbers(
            offset_dims=(), collapsed_slice_dims=(0,), start_index_map=(0,))
        for g in range(K // 16):
            iv = inv1[pl.ds(g * 16, 16)]
            def scale_body(l, cc, iv=iv, g=g):
                bc = lax.gather(iv, jnp.full((16, 1), l, dtype=jnp.int32),
                                dimension_numbers=dnums, slice_sizes=(1,),
                                mode=lax.GatherScatterMode.PROMISE_IN_BOUNDS)
                j = g * 16 + l
                for v in range(EMB // 16):
                    rows[j, pl.ds(v * 16, 16)] = rows[j, pl.ds(v * 16, 16)] * bc
                return cc
            lax.fori_loop(0, 16, scale_body, 0)
        for q in range(QB):
            pltpu.sync_copy(rows.at[pl.ds(q * CHUNK, CHUNK)],
                            acc.at[src2.at[q]], add=True)
        return carry

    lax.fori_loop(0, NB2, p2_body, 0)
    plsc.subcore_barrier()

    pltpu.sync_copy(acc.at[pl.ds(s * TROWS, TROWS)],
                    out_hbm.at[pl.ds(c * NPAD + s * TROWS, TROWS)])


@jax.jit
def kernel(x, W, edge_src, edge_rel, edge_dst):
    BN = 1000
    y = pl.pallas_call(
        _mm_body,
        grid=(R, N // BN),
        in_specs=[pl.BlockSpec((BN, EMB), lambda r, n: (n, 0)),
                  pl.BlockSpec((1, EMB, EMB), lambda r, n: (r, 0, 0))],
        out_specs=pl.BlockSpec((BN, EMB), lambda r, n: (r * (N // BN) + n, 0)),
        out_shape=jax.ShapeDtypeStruct((R * N, EMB), jnp.float32),
    )(x, W)

    pad = E_PAD - E
    srcp = jnp.concatenate(
        [edge_src.astype(jnp.int32), jnp.full((pad,), DUMMY, jnp.int32)])
    relp = jnp.concatenate(
        [edge_rel.astype(jnp.int32), jnp.zeros((pad,), jnp.int32)])
    dstp = jnp.concatenate(
        [edge_dst.astype(jnp.int32), jnp.zeros((pad,), jnp.int32)])
    zacc = jnp.zeros((NPAD, EMB), jnp.float32)
    zcnt = jnp.zeros((CTBL,), jnp.float32)

    mesh = plsc.VectorSubcoreMesh(core_axis_name="c", subcore_axis_name="s")
    accs = pl.kernel(
        _sc_body,
        out_type=jax.ShapeDtypeStruct((NCORE * NPAD, EMB), jnp.float32),
        mesh=mesh,
        scratch_types=[
            pltpu.VMEM((K,), jnp.int32),
            pltpu.VMEM((K,), jnp.int32),
            pltpu.VMEM((K,), jnp.int32),
            pltpu.VMEM((QB, CHUNK), jnp.int32),
            pltpu.VMEM((QB, CHUNK), jnp.int32),
            pltpu.VMEM((QB, CHUNK), jnp.int32),
            pltpu.VMEM((QB, CHUNK), jnp.float32),
            pltpu.VMEM((K,), jnp.float32),
            pltpu.VMEM((K, EMB), jnp.float32),
            pltpu.VMEM((CHUNK,), jnp.float32),
            pltpu.VMEM_SHARED((NPAD, EMB), jnp.float32),
            pltpu.VMEM_SHARED((CTBL,), jnp.float32),
            pltpu.SemaphoreType.DMA,
        ],
    )(y, srcp, relp, dstp, zacc, zcnt)

    BN3 = NPAD // 16
    out = pl.pallas_call(
        _relu_add_body,
        grid=(NPAD // BN3,),
        in_specs=[pl.BlockSpec((BN3, EMB), lambda n: (n, 0)),
                  pl.BlockSpec((BN3, EMB), lambda n: (n + NPAD // BN3, 0))],
        out_specs=pl.BlockSpec((BN3, EMB), lambda n: (n, 0)),
        out_shape=jax.ShapeDtypeStruct((NPAD, EMB), jnp.float32),
    )(accs, accs)
    return out[:N]


# speedup vs baseline: 6.5277x; 1.3971x over previous
import functools

import jax
import jax.numpy as jnp
from jax import lax
from jax.experimental import pallas as pl
from jax.experimental.pallas import tpu as pltpu
from jax.experimental.pallas import tpu_sc as plsc

N = 10000
R = 8
EMB = 128
E = 320000

NSUB = 16
NCORE = 2
K = 96

E_PAD = 322560
DUMMY = N
NPAD = 10112
CSTR = 10240
CTBL = R * CSTR

NB1 = E_PAD // NSUB // K
NB2 = E_PAD // NCORE // NSUB // K

TROWS = NPAD // NSUB
CSTRIPE = CTBL // NSUB

_DNUMS = lax.GatherDimensionNumbers(
    offset_dims=(), collapsed_slice_dims=(0,), start_index_map=(0,))


def _mm_body(x_ref, w_ref, y_ref):
    y_ref[...] = lax.dot_general(
        x_ref[...], w_ref[0],
        dimension_numbers=(((1,), (1,)), ((), ())),
        preferred_element_type=jnp.float32)


def _relu_add_body(a_ref, b_ref, o_ref):
    o_ref[...] = jnp.maximum(a_ref[...] + b_ref[...], 0.0)


def _sc_body(y_hbm, src_hbm, rel_hbm, dst_hbm, zacc_hbm, zcnt_hbm, out_hbm,
             sbuf, rbuf, gidx2, fr2, src2, cnt2, inv1,
             rows0, rows1, rows2, ones, fr1, acc, cnt,
             g0, g1, g2, c0, c1, c2, s0, s1, s2, p0, p1):
    c = lax.axis_index("c")
    s = lax.axis_index("s")
    rows_l = [rows0, rows1, rows2]
    gsem = [g0, g1, g2]
    csem = [c0, c1, c2]
    ssem = [s0, s1, s2]
    psem = [p0, p1]

    pltpu.sync_copy(zacc_hbm.at[pl.ds(s * TROWS, TROWS)],
                    acc.at[pl.ds(s * TROWS, TROWS)])
    pltpu.sync_copy(zcnt_hbm.at[pl.ds(s * CSTRIPE, CSTRIPE)],
                    cnt.at[pl.ds(s * CSTRIPE, CSTRIPE)])
    for g in range(K // 16):
        ones[pl.ds(g * 16, 16)] = jnp.full((16,), 1.0, dtype=jnp.float32)
    plsc.subcore_barrier()

    def p1_fire(slot, b):
        base = (s * NB1 + b) * K
        pltpu.sync_copy(src_hbm.at[pl.ds(base, K)], sbuf)
        pltpu.sync_copy(rel_hbm.at[pl.ds(base, K)], rbuf)
        for g in range(K // 16):
            sv = sbuf[pl.ds(g * 16, 16)]
            rv = rbuf[pl.ds(g * 16, 16)]
            fr1[slot, pl.ds(g * 16, 16)] = rv * CSTR + sv
        pltpu.async_copy(ones, cnt.at[fr1.at[slot]], psem[slot], add=True)

    def p1_wait(slot):
        pltpu.make_async_copy(ones, cnt.at[fr1.at[slot]], psem[slot]).wait()

    p1_fire(0, 0)
    p1_fire(1, 1)

    def p1_outer(o, carry):
        for sl in range(2):
            b = o * 2 + sl
            p1_wait(sl)

            @pl.when(b + 2 < NB1)
            def _():
                p1_fire(sl, b + 2)
        return carry

    lax.fori_loop(0, NB1 // 2, p1_outer, 0)
    plsc.subcore_barrier()

    def fire_gather(slot, b):
        base = c * (E_PAD // 2) + (s * NB2 + b) * K
        pltpu.sync_copy(src_hbm.at[pl.ds(base, K)], sbuf)
        pltpu.sync_copy(rel_hbm.at[pl.ds(base, K)], rbuf)
        pltpu.sync_copy(dst_hbm.at[pl.ds(base, K)], fr1.at[0])
        for g in range(K // 16):
            sv = sbuf[pl.ds(g * 16, 16)]
            rv = rbuf[pl.ds(g * 16, 16)]
            dv = fr1[0, pl.ds(g * 16, 16)]
            gidx2[slot, pl.ds(g * 16, 16)] = rv * N + dv
            fr2[slot, pl.ds(g * 16, 16)] = rv * CSTR + sv
            src2[slot, pl.ds(g * 16, 16)] = sv
        pltpu.async_copy(y_hbm.at[gidx2.at[slot]], rows_l[slot], gsem[slot])
        pltpu.async_copy(cnt.at[fr2.at[slot]], cnt2.at[slot], csem[slot])

    def wait_gather(slot):
        pltpu.make_async_copy(
            y_hbm.at[gidx2.at[slot]], rows_l[slot], gsem[slot]).wait()
        pltpu.make_async_copy(
            cnt.at[fr2.at[slot]], cnt2.at[slot], csem[slot]).wait()

    def fire_scatter(slot):
        pltpu.async_copy(rows_l[slot], acc.at[src2.at[slot]], ssem[slot],
                         add=True)

    def wait_scatter(slot):
        pltpu.make_async_copy(
            rows_l[slot], acc.at[src2.at[slot]], ssem[slot]).wait()

    def scale(slot):
        r = rows_l[slot]
        for g in range(K // 16):
            cv = cnt2[slot, pl.ds(g * 16, 16)]
            inv1[pl.ds(g * 16, 16)] = 1.0 / cv
        for g in range(K // 16):
            iv = inv1[pl.ds(g * 16, 16)]
            def scale_body(l, cc, iv=iv, g=g):
                bc = lax.gather(iv, jnp.full((16, 1), l, dtype=jnp.int32),
                                dimension_numbers=_DNUMS, slice_sizes=(1,),
                                mode=lax.GatherScatterMode.PROMISE_IN_BOUNDS)
                j = g * 16 + l
                for v in range(EMB // 16):
                    r[j, pl.ds(v * 16, 16)] = r[j, pl.ds(v * 16, 16)] * bc
                return cc
            lax.fori_loop(0, 16, scale_body, 0)

    fire_gather(0, 0)
    fire_gather(1, 1)

    def p2_outer(o, carry):
        for s3 in range(3):
            b = o * 3 + s3
            wait_gather(s3)
            scale(s3)

            @pl.when(b >= 1)
            def _():
                wait_scatter((s3 + 2) % 3)

            fire_scatter(s3)

            @pl.when(b + 2 < NB2)
            def _():
                fire_gather((s3 + 2) % 3, b + 2)
        return carry

    lax.fori_loop(0, NB2 // 3, p2_outer, 0)
    wait_scatter((NB2 - 1) % 3)
    plsc.subcore_barrier()

    pltpu.sync_copy(acc.at[pl.ds(s * TROWS, TROWS)],
                    out_hbm.at[pl.ds(c * NPAD + s * TROWS, TROWS)])


@jax.jit
def kernel(x, W, edge_src, edge_rel, edge_dst):
    BN = 1000
    y = pl.pallas_call(
        _mm_body,
        grid=(R, N // BN),
        in_specs=[pl.BlockSpec((BN, EMB), lambda r, n: (n, 0)),
                  pl.BlockSpec((1, EMB, EMB), lambda r, n: (r, 0, 0))],
        out_specs=pl.BlockSpec((BN, EMB), lambda r, n: (r * (N // BN) + n, 0)),
        out_shape=jax.ShapeDtypeStruct((R * N, EMB), jnp.float32),
    )(x, W)

    pad = E_PAD - E
    srcp = jnp.concatenate(
        [edge_src.astype(jnp.int32), jnp.full((pad,), DUMMY, jnp.int32)])
    relp = jnp.concatenate(
        [edge_rel.astype(jnp.int32), jnp.zeros((pad,), jnp.int32)])
    dstp = jnp.concatenate(
        [edge_dst.astype(jnp.int32), jnp.zeros((pad,), jnp.int32)])
    zacc = jnp.zeros((NPAD, EMB), jnp.float32)
    zcnt = jnp.zeros((CTBL,), jnp.float32)

    mesh = plsc.VectorSubcoreMesh(core_axis_name="c", subcore_axis_name="s")
    accs = pl.kernel(
        _sc_body,
        out_type=jax.ShapeDtypeStruct((NCORE * NPAD, EMB), jnp.float32),
        mesh=mesh,
        scratch_types=[
            pltpu.VMEM((K,), jnp.int32),
            pltpu.VMEM((K,), jnp.int32),
            pltpu.VMEM((3, K), jnp.int32),
            pltpu.VMEM((3, K), jnp.int32),
            pltpu.VMEM((3, K), jnp.int32),
            pltpu.VMEM((3, K), jnp.float32),
            pltpu.VMEM((K,), jnp.float32),
            pltpu.VMEM((K, EMB), jnp.float32),
            pltpu.VMEM((K, EMB), jnp.float32),
            pltpu.VMEM((K, EMB), jnp.float32),
            pltpu.VMEM((K,), jnp.float32),
            pltpu.VMEM((2, K), jnp.int32),
            pltpu.VMEM_SHARED((NPAD, EMB), jnp.float32),
            pltpu.VMEM_SHARED((CTBL,), jnp.float32),
            pltpu.SemaphoreType.DMA,
            pltpu.SemaphoreType.DMA,
            pltpu.SemaphoreType.DMA,
            pltpu.SemaphoreType.DMA,
            pltpu.SemaphoreType.DMA,
            pltpu.SemaphoreType.DMA,
            pltpu.SemaphoreType.DMA,
            pltpu.SemaphoreType.DMA,
            pltpu.SemaphoreType.DMA,
            pltpu.SemaphoreType.DMA,
            pltpu.SemaphoreType.DMA,
        ],
    )(y, srcp, relp, dstp, zacc, zcnt)

    BN3 = NPAD // 4
    out = pl.pallas_call(
        _relu_add_body,
        grid=(NPAD // BN3,),
        in_specs=[pl.BlockSpec((BN3, EMB), lambda n: (n, 0)),
                  pl.BlockSpec((BN3, EMB), lambda n: (n + NPAD // BN3, 0))],
        out_specs=pl.BlockSpec((BN3, EMB), lambda n: (n, 0)),
        out_shape=jax.ShapeDtypeStruct((NPAD, EMB), jnp.float32),
    )(accs, accs)
    return out[:N]


# speedup vs baseline: 9.1383x; 1.3999x over previous
import functools

import jax
import jax.numpy as jnp
from jax import lax
from jax.experimental import pallas as pl
from jax.experimental.pallas import tpu as pltpu
from jax.experimental.pallas import tpu_sc as plsc

N = 10000
R = 8
EMB = 128
E = 320000

NSUB = 16
NCORE = 2
K = 96

E_PAD = 322560
DUMMY = N
NPAD = 10112
CSTR = 10240
CTBL = R * CSTR

NB1 = E_PAD // NSUB // K
NB2 = E_PAD // NCORE // NSUB // K

TROWS = NPAD // NSUB
CSTRIPE = CTBL // NSUB
RCH = 1024

_DNUMS = lax.GatherDimensionNumbers(
    offset_dims=(), collapsed_slice_dims=(0,), start_index_map=(0,))


def _mm_body(x_ref, w_ref, y_ref):
    y_ref[...] = lax.dot_general(
        x_ref[...], w_ref[0],
        dimension_numbers=(((1,), (1,)), ((), ())),
        preferred_element_type=jnp.float32)


def _relu_add_body(a_ref, b_ref, o_ref):
    o_ref[...] = jnp.maximum(a_ref[...] + b_ref[...], 0.0)


def _sc_body(y_hbm, src_hbm, rel_hbm, dst_hbm, zacc_hbm, zcnt_hbm, out_hbm,
             sb2, rb2, db2, gidx2, fr2, src2, cnt2, cbuf,
             rows0, rows1, rows2, ones, fr1, acc, cnt,
             i0, i1, i2, g0, g1, g2, c0, c1, c2, s0, s1, s2, p0, p1):
    c = lax.axis_index("c")
    s = lax.axis_index("s")
    rows_l = [rows0, rows1, rows2]
    isem = [i0, i1, i2]
    gsem = [g0, g1, g2]
    csem = [c0, c1, c2]
    ssem = [s0, s1, s2]
    psem = [p0, p1]

    pltpu.sync_copy(zacc_hbm, acc.at[pl.ds(s * TROWS, TROWS)])
    pltpu.sync_copy(zcnt_hbm, cnt.at[pl.ds(s * CSTRIPE, CSTRIPE)])
    for g in range(K // 16):
        ones[pl.ds(g * 16, 16)] = jnp.full((16,), 1.0, dtype=jnp.float32)
    plsc.subcore_barrier()

    def p1_load(slot, b):
        base = (s * NB1 + b) * K
        pltpu.async_copy(src_hbm.at[pl.ds(base, K)], sb2.at[slot], isem[slot])
        pltpu.async_copy(rel_hbm.at[pl.ds(base, K)], rb2.at[slot], isem[slot])

    def p1_wait_load(slot, b):
        base = (s * NB1 + b) * K
        pltpu.make_async_copy(
            src_hbm.at[pl.ds(base, K)], sb2.at[slot], isem[slot]).wait()
        pltpu.make_async_copy(
            rel_hbm.at[pl.ds(base, K)], rb2.at[slot], isem[slot]).wait()

    def p1_wait_scatter(slot):
        pltpu.make_async_copy(ones, cnt.at[fr1.at[slot]], psem[slot]).wait()

    p1_load(0, 0)
    p1_load(1, 1)

    def p1_outer(o, carry):
        for sl in range(2):
            b = o * 2 + sl

            @pl.when(b >= 2)
            def _():
                p1_wait_scatter(sl)

            p1_wait_load(sl, b)
            for g in range(K // 16):
                sv = sb2[sl, pl.ds(g * 16, 16)]
                rv = rb2[sl, pl.ds(g * 16, 16)]
                fr1[sl, pl.ds(g * 16, 16)] = rv * CSTR + sv
            pltpu.async_copy(ones, cnt.at[fr1.at[sl]], psem[sl], add=True)

            @pl.when(b + 2 < NB1)
            def _():
                p1_load(sl, b + 2)
        return carry

    lax.fori_loop(0, NB1 // 2, p1_outer, 0)
    p1_wait_scatter(0)
    p1_wait_scatter(1)
    plsc.subcore_barrier()

    def recip_chunk(i, carry):
        base = s * CSTRIPE + i * RCH
        pltpu.sync_copy(cnt.at[pl.ds(base, RCH)], cbuf)
        for g in range(RCH // 16):
            cbuf[pl.ds(g * 16, 16)] = 1.0 / cbuf[pl.ds(g * 16, 16)]
        pltpu.sync_copy(cbuf, cnt.at[pl.ds(base, RCH)])
        return carry

    lax.fori_loop(0, CSTRIPE // RCH, recip_chunk, 0)
    plsc.subcore_barrier()

    def p2_load(slot, b):
        base = c * (E_PAD // 2) + (s * NB2 + b) * K
        pltpu.async_copy(src_hbm.at[pl.ds(base, K)], sb2.at[slot], isem[slot])
        pltpu.async_copy(rel_hbm.at[pl.ds(base, K)], rb2.at[slot], isem[slot])
        pltpu.async_copy(dst_hbm.at[pl.ds(base, K)], db2.at[slot], isem[slot])

    def p2_wait_load(slot, b):
        base = c * (E_PAD // 2) + (s * NB2 + b) * K
        pltpu.make_async_copy(
            src_hbm.at[pl.ds(base, K)], sb2.at[slot], isem[slot]).wait()
        pltpu.make_async_copy(
            rel_hbm.at[pl.ds(base, K)], rb2.at[slot], isem[slot]).wait()
        pltpu.make_async_copy(
            dst_hbm.at[pl.ds(base, K)], db2.at[slot], isem[slot]).wait()

    def fire_gather(slot):
        for g in range(K // 16):
            sv = sb2[slot, pl.ds(g * 16, 16)]
            rv = rb2[slot, pl.ds(g * 16, 16)]
            dv = db2[slot, pl.ds(g * 16, 16)]
            gidx2[slot, pl.ds(g * 16, 16)] = rv * N + dv
            fr2[slot, pl.ds(g * 16, 16)] = rv * CSTR + sv
            src2[slot, pl.ds(g * 16, 16)] = sv
        pltpu.async_copy(y_hbm.at[gidx2.at[slot]], rows_l[slot], gsem[slot])
        pltpu.async_copy(cnt.at[fr2.at[slot]], cnt2.at[slot], csem[slot])

    def wait_gather(slot):
        pltpu.make_async_copy(
            y_hbm.at[gidx2.at[slot]], rows_l[slot], gsem[slot]).wait()
        pltpu.make_async_copy(
            cnt.at[fr2.at[slot]], cnt2.at[slot], csem[slot]).wait()

    def fire_scatter(slot):
        pltpu.async_copy(rows_l[slot], acc.at[src2.at[slot]], ssem[slot],
                         add=True)

    def wait_scatter(slot):
        pltpu.make_async_copy(
            rows_l[slot], acc.at[src2.at[slot]], ssem[slot]).wait()

    def scale(slot):
        r = rows_l[slot]
        for g in range(K // 16):
            iv = cnt2[slot, pl.ds(g * 16, 16)]
            for l in range(16):
                bc = lax.gather(iv, jnp.full((16, 1), l, dtype=jnp.int32),
                                dimension_numbers=_DNUMS, slice_sizes=(1,),
                                mode=lax.GatherScatterMode.PROMISE_IN_BOUNDS)
                j = g * 16 + l
                for v in range(EMB // 16):
                    r[j, pl.ds(v * 16, 16)] = r[j, pl.ds(v * 16, 16)] * bc

    p2_load(0, 0)
    p2_load(1, 1)
    p2_load(2, 2)
    p2_wait_load(0, 0)
    fire_gather(0)
    p2_wait_load(1, 1)
    fire_gather(1)

    def p2_outer(o, carry):
        for s3 in range(3):
            b = o * 3 + s3
            wait_gather(s3)
            scale(s3)

            @pl.when(b >= 1)
            def _():
                wait_scatter((s3 + 2) % 3)

            fire_scatter(s3)

            @pl.when(b + 3 < NB2)
            def _():
                p2_load(s3, b + 3)

            @pl.when(b + 2 < NB2)
            def _():
                p2_wait_load((s3 + 2) % 3, b + 2)
                fire_gather((s3 + 2) % 3)
        return carry

    lax.fori_loop(0, NB2 // 3, p2_outer, 0)
    wait_scatter((NB2 - 1) % 3)
    plsc.subcore_barrier()

    pltpu.sync_copy(acc.at[pl.ds(s * TROWS, TROWS)],
                    out_hbm.at[pl.ds(c * NPAD + s * TROWS, TROWS)])


@jax.jit
def kernel(x, W, edge_src, edge_rel, edge_dst):
    BN = 1000
    y = pl.pallas_call(
        _mm_body,
        grid=(R, N // BN),
        in_specs=[pl.BlockSpec((BN, EMB), lambda r, n: (n, 0)),
                  pl.BlockSpec((1, EMB, EMB), lambda r, n: (r, 0, 0))],
        out_specs=pl.BlockSpec((BN, EMB), lambda r, n: (r * (N // BN) + n, 0)),
        out_shape=jax.ShapeDtypeStruct((R * N, EMB), jnp.float32),
    )(x, W)

    pad = E_PAD - E
    srcp = jnp.concatenate(
        [edge_src.astype(jnp.int32), jnp.full((pad,), DUMMY, jnp.int32)])
    relp = jnp.concatenate(
        [edge_rel.astype(jnp.int32), jnp.zeros((pad,), jnp.int32)])
    dstp = jnp.concatenate(
        [edge_dst.astype(jnp.int32), jnp.zeros((pad,), jnp.int32)])
    zacc = jnp.zeros((TROWS, EMB), jnp.float32)
    zcnt = jnp.zeros((CSTRIPE,), jnp.float32)

    mesh = plsc.VectorSubcoreMesh(core_axis_name="c", subcore_axis_name="s")
    accs = pl.kernel(
        _sc_body,
        out_type=jax.ShapeDtypeStruct((NCORE * NPAD, EMB), jnp.float32),
        mesh=mesh,
        scratch_types=[
            pltpu.VMEM((3, K), jnp.int32),
            pltpu.VMEM((3, K), jnp.int32),
            pltpu.VMEM((3, K), jnp.int32),
            pltpu.VMEM((3, K), jnp.int32),
            pltpu.VMEM((3, K), jnp.int32),
            pltpu.VMEM((3, K), jnp.int32),
            pltpu.VMEM((3, K), jnp.float32),
            pltpu.VMEM((RCH,), jnp.float32),
            pltpu.VMEM((K, EMB), jnp.float32),
            pltpu.VMEM((K, EMB), jnp.float32),
            pltpu.VMEM((K, EMB), jnp.float32),
            pltpu.VMEM((K,), jnp.float32),
            pltpu.VMEM((2, K), jnp.int32),
            pltpu.VMEM_SHARED((NPAD, EMB), jnp.float32),
            pltpu.VMEM_SHARED((CTBL,), jnp.float32),
            pltpu.SemaphoreType.DMA,
            pltpu.SemaphoreType.DMA,
            pltpu.SemaphoreType.DMA,
            pltpu.SemaphoreType.DMA,
            pltpu.SemaphoreType.DMA,
            pltpu.SemaphoreType.DMA,
            pltpu.SemaphoreType.DMA,
            pltpu.SemaphoreType.DMA,
            pltpu.SemaphoreType.DMA,
            pltpu.SemaphoreType.DMA,
            pltpu.SemaphoreType.DMA,
            pltpu.SemaphoreType.DMA,
            pltpu.SemaphoreType.DMA,
            pltpu.SemaphoreType.DMA,
        ],
    )(y, srcp, relp, dstp, zacc, zcnt)

    BN3 = NPAD // 4
    out = pl.pallas_call(
        _relu_add_body,
        grid=(NPAD // BN3,),
        in_specs=[pl.BlockSpec((BN3, EMB), lambda n: (n, 0)),
                  pl.BlockSpec((BN3, EMB), lambda n: (n + NPAD // BN3, 0))],
        out_specs=pl.BlockSpec((BN3, EMB), lambda n: (n, 0)),
        out_shape=jax.ShapeDtypeStruct((NPAD, EMB), jnp.float32),
    )(accs, accs)
    return out[:N]


# speedup vs baseline: 9.4070x; 1.0294x over previous
import functools

import jax
import jax.numpy as jnp
from jax import lax
from jax.experimental import pallas as pl
from jax.experimental.pallas import tpu as pltpu
from jax.experimental.pallas import tpu_sc as plsc

N = 10000
R = 8
EMB = 128
E = 320000

NSUB = 16
NCORE = 2
K = 96

E_PAD = 322560
DUMMY = N
NPAD = 10112
CSTR = 10240
CTBL = R * CSTR

NB1 = E_PAD // NSUB // K
NB2 = E_PAD // NCORE // NSUB // K

TROWS = NPAD // NSUB
CSTRIPE = CTBL // NSUB
RCH = CSTRIPE

_DNUMS = lax.GatherDimensionNumbers(
    offset_dims=(), collapsed_slice_dims=(0,), start_index_map=(0,))


def _mm_body(x_ref, w_ref, y_ref):
    y_ref[...] = lax.dot_general(
        x_ref[...], w_ref[0],
        dimension_numbers=(((1,), (1,)), ((), ())),
        preferred_element_type=jnp.float32)


def _relu_add_body(a_ref, b_ref, o_ref):
    o_ref[...] = jnp.maximum(a_ref[...] + b_ref[...], 0.0)


def _sc_body(y_hbm, cidx_hbm, gidx_hbm, src_hbm, zacc_hbm, zcnt_hbm, out_hbm,
             gb2, cb2, db2, src2, cnt2, cbuf,
             rows0, rows1, rows2, ones, fr1, acc, cnt,
             i0, i1, i2, g0, g1, g2, c0, c1, c2, s0, s1, s2, p0, p1):
    c = lax.axis_index("c")
    s = lax.axis_index("s")
    rows_l = [rows0, rows1, rows2]
    isem = [i0, i1, i2]
    gsem = [g0, g1, g2]
    csem = [c0, c1, c2]
    ssem = [s0, s1, s2]
    psem = [p0, p1]

    pltpu.sync_copy(zacc_hbm, acc.at[pl.ds(s * TROWS, TROWS)])
    pltpu.sync_copy(zcnt_hbm, cnt.at[pl.ds(s * CSTRIPE, CSTRIPE)])
    for g in range(K // 16):
        ones[pl.ds(g * 16, 16)] = jnp.full((16,), 1.0, dtype=jnp.float32)
    plsc.subcore_barrier()

    def p1_load(slot, b):
        base = (s * NB1 + b) * K
        pltpu.async_copy(cidx_hbm.at[pl.ds(base, K)], cb2.at[slot], isem[slot])

    def p1_wait_load(slot, b):
        base = (s * NB1 + b) * K
        pltpu.make_async_copy(
            cidx_hbm.at[pl.ds(base, K)], cb2.at[slot], isem[slot]).wait()

    def p1_wait_scatter(slot):
        pltpu.make_async_copy(ones, cnt.at[fr1.at[slot]], psem[slot]).wait()

    p1_load(0, 0)
    p1_load(1, 1)

    def p1_outer(o, carry):
        for sl in range(2):
            b = o * 2 + sl

            @pl.when(b >= 2)
            def _():
                p1_wait_scatter(sl)

            p1_wait_load(sl, b)
            for g in range(K // 16):
                fr1[sl, pl.ds(g * 16, 16)] = cb2[sl, pl.ds(g * 16, 16)]
            pltpu.async_copy(ones, cnt.at[fr1.at[sl]], psem[sl], add=True)

            @pl.when(b + 2 < NB1)
            def _():
                p1_load(sl, b + 2)
        return carry

    lax.fori_loop(0, NB1 // 2, p1_outer, 0)
    p1_wait_scatter(0)
    p1_wait_scatter(1)
    plsc.subcore_barrier()

    def recip_chunk(i, carry):
        base = s * CSTRIPE + i * RCH
        pltpu.sync_copy(cnt.at[pl.ds(base, RCH)], cbuf)
        for g in range(RCH // 16):
            cbuf[pl.ds(g * 16, 16)] = 1.0 / cbuf[pl.ds(g * 16, 16)]
        pltpu.sync_copy(cbuf, cnt.at[pl.ds(base, RCH)])
        return carry

    lax.fori_loop(0, CSTRIPE // RCH, recip_chunk, 0)
    plsc.subcore_barrier()

    def p2_load(slot, b):
        base = c * (E_PAD // 2) + (s * NB2 + b) * K
        pltpu.async_copy(gidx_hbm.at[pl.ds(base, K)], gb2.at[slot], isem[slot])
        pltpu.async_copy(cidx_hbm.at[pl.ds(base, K)], cb2.at[slot], isem[slot])
        pltpu.async_copy(src_hbm.at[pl.ds(base, K)], db2.at[slot], isem[slot])

    def p2_wait_load(slot, b):
        base = c * (E_PAD // 2) + (s * NB2 + b) * K
        pltpu.make_async_copy(
            gidx_hbm.at[pl.ds(base, K)], gb2.at[slot], isem[slot]).wait()
        pltpu.make_async_copy(
            cidx_hbm.at[pl.ds(base, K)], cb2.at[slot], isem[slot]).wait()
        pltpu.make_async_copy(
            src_hbm.at[pl.ds(base, K)], db2.at[slot], isem[slot]).wait()

    def fire_gather(slot):
        for g in range(K // 16):
            src2[slot, pl.ds(g * 16, 16)] = db2[slot, pl.ds(g * 16, 16)]
        pltpu.async_copy(y_hbm.at[gb2.at[slot]], rows_l[slot], gsem[slot])
        pltpu.async_copy(cnt.at[cb2.at[slot]], cnt2.at[slot], csem[slot])

    def wait_gather(slot):
        pltpu.make_async_copy(
            y_hbm.at[gb2.at[slot]], rows_l[slot], gsem[slot]).wait()
        pltpu.make_async_copy(
            cnt.at[cb2.at[slot]], cnt2.at[slot], csem[slot]).wait()

    def fire_scatter(slot):
        pltpu.async_copy(rows_l[slot], acc.at[src2.at[slot]], ssem[slot],
                         add=True)

    def wait_scatter(slot):
        pltpu.make_async_copy(
            rows_l[slot], acc.at[src2.at[slot]], ssem[slot]).wait()

    def scale(slot):
        r = rows_l[slot]
        for g in range(K // 16):
            iv = cnt2[slot, pl.ds(g * 16, 16)]
            for l in range(16):
                bc = lax.gather(iv, jnp.full((16, 1), l, dtype=jnp.int32),
                                dimension_numbers=_DNUMS, slice_sizes=(1,),
                                mode=lax.GatherScatterMode.PROMISE_IN_BOUNDS)
                j = g * 16 + l
                for v in range(EMB // 16):
                    r[j, pl.ds(v * 16, 16)] = r[j, pl.ds(v * 16, 16)] * bc

    p2_load(0, 0)
    p2_load(1, 1)
    p2_load(2, 2)
    p2_wait_load(0, 0)
    fire_gather(0)
    p2_wait_load(1, 1)
    fire_gather(1)

    def p2_outer(o, carry):
        for s3 in range(3):
            b = o * 3 + s3
            wait_gather(s3)
            scale(s3)

            @pl.when(b >= 1)
            def _():
                wait_scatter((s3 + 2) % 3)

            fire_scatter(s3)

            @pl.when(b + 3 < NB2)
            def _():
                p2_load(s3, b + 3)

            @pl.when(b + 2 < NB2)
            def _():
                p2_wait_load((s3 + 2) % 3, b + 2)
                fire_gather((s3 + 2) % 3)
        return carry

    lax.fori_loop(0, NB2 // 3, p2_outer, 0)
    wait_scatter((NB2 - 1) % 3)
    plsc.subcore_barrier()

    pltpu.sync_copy(acc.at[pl.ds(s * TROWS, TROWS)],
                    out_hbm.at[pl.ds(c * NPAD + s * TROWS, TROWS)])


@jax.jit
def kernel(x, W, edge_src, edge_rel, edge_dst):
    BN = 1000
    y = pl.pallas_call(
        _mm_body,
        grid=(R, N // BN),
        in_specs=[pl.BlockSpec((BN, EMB), lambda r, n: (n, 0)),
                  pl.BlockSpec((1, EMB, EMB), lambda r, n: (r, 0, 0))],
        out_specs=pl.BlockSpec((BN, EMB), lambda r, n: (r * (N // BN) + n, 0)),
        out_shape=jax.ShapeDtypeStruct((R * N, EMB), jnp.float32),
    )(x, W)

    pad = E_PAD - E
    srcp = jnp.concatenate(
        [edge_src.astype(jnp.int32), jnp.full((pad,), DUMMY, jnp.int32)])
    relp = jnp.concatenate(
        [edge_rel.astype(jnp.int32), jnp.zeros((pad,), jnp.int32)])
    dstp = jnp.concatenate(
        [edge_dst.astype(jnp.int32), jnp.zeros((pad,), jnp.int32)])
    cidxp = relp * CSTR + srcp
    gidxp = relp * N + dstp
    zacc = jnp.zeros((TROWS, EMB), jnp.float32)
    zcnt = jnp.zeros((CSTRIPE,), jnp.float32)

    mesh = plsc.VectorSubcoreMesh(core_axis_name="c", subcore_axis_name="s")
    accs = pl.kernel(
        _sc_body,
        out_type=jax.ShapeDtypeStruct((NCORE * NPAD, EMB), jnp.float32),
        mesh=mesh,
        scratch_types=[
            pltpu.VMEM((3, K), jnp.int32),
            pltpu.VMEM((3, K), jnp.int32),
            pltpu.VMEM((3, K), jnp.int32),
            pltpu.VMEM((3, K), jnp.int32),
            pltpu.VMEM((3, K), jnp.float32),
            pltpu.VMEM((RCH,), jnp.float32),
            pltpu.VMEM((K, EMB), jnp.float32),
            pltpu.VMEM((K, EMB), jnp.float32),
            pltpu.VMEM((K, EMB), jnp.float32),
            pltpu.VMEM((K,), jnp.float32),
            pltpu.VMEM((2, K), jnp.int32),
            pltpu.VMEM_SHARED((NPAD, EMB), jnp.float32),
            pltpu.VMEM_SHARED((CTBL,), jnp.float32),
            pltpu.SemaphoreType.DMA,
            pltpu.SemaphoreType.DMA,
            pltpu.SemaphoreType.DMA,
            pltpu.SemaphoreType.DMA,
            pltpu.SemaphoreType.DMA,
            pltpu.SemaphoreType.DMA,
            pltpu.SemaphoreType.DMA,
            pltpu.SemaphoreType.DMA,
            pltpu.SemaphoreType.DMA,
            pltpu.SemaphoreType.DMA,
            pltpu.SemaphoreType.DMA,
            pltpu.SemaphoreType.DMA,
            pltpu.SemaphoreType.DMA,
            pltpu.SemaphoreType.DMA,
        ],
    )(y, cidxp, gidxp, srcp, zacc, zcnt)

    BN3 = NPAD // 4
    out = pl.pallas_call(
        _relu_add_body,
        grid=(NPAD // BN3,),
        in_specs=[pl.BlockSpec((BN3, EMB), lambda n: (n, 0)),
                  pl.BlockSpec((BN3, EMB), lambda n: (n + NPAD // BN3, 0))],
        out_specs=pl.BlockSpec((BN3, EMB), lambda n: (n, 0)),
        out_shape=jax.ShapeDtypeStruct((NPAD, EMB), jnp.float32),
    )(accs, accs)
    return out[:N]
